# Initial kernel scaffold; baseline (speedup 1.0000x reference)
import jax
import jax.numpy as jnp
from jax.experimental import pallas as pl


def kernel(x, edge_index, W, b, gamma, beta):
    raise NotImplementedError("write your pallas kernel here")


# speedup vs baseline: 6.1006x; 6.1006x over previous
import functools

import jax
import jax.numpy as jnp
from jax import lax
from jax.experimental import pallas as pl
from jax.experimental.pallas import tpu as pltpu
from jax.experimental.pallas import tpu_sc as plsc

N = 10000
E = 320000
D = 128

NC = 2
NS = 16
NW = NC * NS

CHUNK = 128
N_PAD = 10240
ROWS_PER_TILE = N_PAD // NS
E_PAD = ((E + NW * CHUNK - 1) // (NW * CHUNK)) * (NW * CHUNK)
EPT = E_PAD // NW
NCHUNKS = EPT // CHUNK

_mesh = plsc.VectorSubcoreMesh(core_axis_name="c", subcore_axis_name="s")


@functools.partial(
    pl.kernel,
    mesh=_mesh,
    out_type=jax.ShapeDtypeStruct((2, NC, N_PAD), jnp.float32),
    scratch_types=[
        pltpu.VMEM((CHUNK,), jnp.int32),
        pltpu.VMEM((CHUNK,), jnp.float32),
        pltpu.VMEM_SHARED((N_PAD,), jnp.float32),
        pltpu.VMEM_SHARED((N_PAD,), jnp.float32),
    ],
)
def _sc_degrees(src_hbm, dst_hbm, ones_hbm, zeros_hbm, out_hbm,
                idx_v, ones_v, dego_sp, degi_sp):
    c = lax.axis_index("c")
    s = lax.axis_index("s")
    wid = s * NC + c
    pltpu.sync_copy(zeros_hbm, dego_sp.at[pl.ds(s * ROWS_PER_TILE, ROWS_PER_TILE)])
    pltpu.sync_copy(zeros_hbm, degi_sp.at[pl.ds(s * ROWS_PER_TILE, ROWS_PER_TILE)])
    pltpu.sync_copy(ones_hbm, ones_v)
    plsc.subcore_barrier()

    def body(i, carry):
        base = wid * EPT + i * CHUNK
        pltpu.sync_copy(src_hbm.at[pl.ds(base, CHUNK)], idx_v)
        pltpu.sync_copy(ones_v, dego_sp.at[idx_v], add=True)
        pltpu.sync_copy(dst_hbm.at[pl.ds(base, CHUNK)], idx_v)
        pltpu.sync_copy(ones_v, degi_sp.at[idx_v], add=True)
        return carry

    lax.fori_loop(0, NCHUNKS, body, 0)
    plsc.subcore_barrier()
    sl = pl.ds(s * ROWS_PER_TILE, ROWS_PER_TILE)
    pltpu.sync_copy(dego_sp.at[sl], out_hbm.at[0, c, sl])
    pltpu.sync_copy(degi_sp.at[sl], out_hbm.at[1, c, sl])


@functools.partial(
    pl.kernel,
    mesh=_mesh,
    out_type=jax.ShapeDtypeStruct((NC, N_PAD, D), jnp.float32),
    scratch_types=[
        pltpu.VMEM((CHUNK,), jnp.int32),
        pltpu.VMEM((CHUNK,), jnp.int32),
        pltpu.VMEM((CHUNK, D), jnp.float32),
        pltpu.VMEM_SHARED((N_PAD, D), jnp.float32),
        pltpu.SemaphoreType.DMA,
    ],
)
def _sc_gather_scatter(h_hbm, src_hbm, dst_hbm, zrows_hbm, out_hbm,
                       idx_s, idx_d, rows_v, agg_sp, sem):
    c = lax.axis_index("c")
    s = lax.axis_index("s")
    wid = s * NC + c
    sl = pl.ds(s * ROWS_PER_TILE, ROWS_PER_TILE)
    pltpu.sync_copy(zrows_hbm, agg_sp.at[sl])
    plsc.subcore_barrier()

    def body(i, carry):
        base = wid * EPT + i * CHUNK
        pltpu.sync_copy(src_hbm.at[pl.ds(base, CHUNK)], idx_s)
        pltpu.sync_copy(dst_hbm.at[pl.ds(base, CHUNK)], idx_d)
        pltpu.async_copy(h_hbm.at[idx_s], rows_v, sem).wait()
        pltpu.sync_copy(rows_v, agg_sp.at[idx_d], add=True)
        return carry

    lax.fori_loop(0, NCHUNKS, body, 0)
    plsc.subcore_barrier()
    pltpu.sync_copy(agg_sp.at[sl], out_hbm.at[c, sl])


def _h_body(x_ref, d0_ref, d1_ref, o_ref):
    d = d0_ref[0, 0, :] + d1_ref[0, 0, :]
    norm = jnp.where(d > 0.0, lax.rsqrt(d), 0.0)
    o_ref[...] = x_ref[...] * norm[:, None]


def _tc_scale(x_pad, degs3d):
    R = 1024
    return pl.pallas_call(
        _h_body,
        grid=(N_PAD // R,),
        in_specs=[
            pl.BlockSpec((R, D), lambda i: (i, 0)),
            pl.BlockSpec((1, 1, R), lambda i: (0, 0, i)),
            pl.BlockSpec((1, 1, R), lambda i: (1, 0, i)),
        ],
        out_specs=pl.BlockSpec((R, D), lambda i: (i, 0)),
        out_shape=jax.ShapeDtypeStruct((N_PAD, D), jnp.float32),
    )(x_pad, degs3d, degs3d)


def _dense_body(a0_ref, a1_ref, d0_ref, d1_ref, x_ref, w_ref, b_ref,
                g_ref, bt_ref, o_ref):
    agg = a0_ref[0] + a1_ref[0]
    din = d0_ref[0, 0, :] + d1_ref[0, 0, :]
    norm = jnp.where(din > 0.0, lax.rsqrt(din), 0.0)
    a = agg * norm[:, None]
    o = jnp.dot(a, w_ref[...], preferred_element_type=jnp.float32) + b_ref[0]
    mu = jnp.mean(o, axis=1, keepdims=True)
    var = jnp.mean((o - mu) ** 2, axis=1, keepdims=True)
    o = (o - mu) * lax.rsqrt(var + 1e-5) * g_ref[0] + bt_ref[0]
    o_ref[...] = jnp.maximum(o, 0.0) + x_ref[...]


def _tc_dense(agg_p, degs3d, x_pad, W, b, gamma, beta):
    R = 1024
    return pl.pallas_call(
        _dense_body,
        grid=(N_PAD // R,),
        in_specs=[
            pl.BlockSpec((1, R, D), lambda i: (0, i, 0)),
            pl.BlockSpec((1, R, D), lambda i: (1, i, 0)),
            pl.BlockSpec((1, 1, R), lambda i: (2, 0, i)),
            pl.BlockSpec((1, 1, R), lambda i: (3, 0, i)),
            pl.BlockSpec((R, D), lambda i: (i, 0)),
            pl.BlockSpec((D, D), lambda i: (0, 0)),
            pl.BlockSpec((1, D), lambda i: (0, 0)),
            pl.BlockSpec((1, D), lambda i: (0, 0)),
            pl.BlockSpec((1, D), lambda i: (0, 0)),
        ],
        out_specs=pl.BlockSpec((R, D), lambda i: (i, 0)),
        out_shape=jax.ShapeDtypeStruct((N_PAD, D), jnp.float32),
    )(agg_p, agg_p, degs3d, degs3d, x_pad, W,
      b.reshape(1, D), gamma.reshape(1, D), beta.reshape(1, D))


def kernel(x, edge_index, W, b, gamma, beta):
    src = edge_index[0].astype(jnp.int32)
    dst = edge_index[1].astype(jnp.int32)
    pad_n = E_PAD - E
    pad_idx = N + (jnp.arange(pad_n, dtype=jnp.int32) % (N_PAD - N))
    src_p = jnp.concatenate([src, pad_idx])
    dst_p = jnp.concatenate([dst, pad_idx])

    ones_c = jnp.ones((CHUNK,), jnp.float32)
    zeros_r = jnp.zeros((ROWS_PER_TILE,), jnp.float32)
    zeros_rows = jnp.zeros((ROWS_PER_TILE, D), jnp.float32)

    degs = _sc_degrees(src_p, dst_p, ones_c, zeros_r)
    degs3d = degs.reshape(2 * NC, 1, N_PAD)

    x_pad = jnp.pad(x, ((0, N_PAD - N), (0, 0)))
    h = _tc_scale(x_pad, degs3d)
    agg_p = _sc_gather_scatter(h, src_p, dst_p, zeros_rows)
    return _tc_dense(agg_p, degs3d, x_pad, W, b, gamma, beta)[:N]


# speedup vs baseline: 12.9488x; 2.1226x over previous
import functools

import jax
import jax.numpy as jnp
from jax import lax
from jax.experimental import pallas as pl
from jax.experimental.pallas import tpu as pltpu
from jax.experimental.pallas import tpu_sc as plsc

N = 10000
E = 320000
D = 128

NC = 2
NS = 16
NW = NC * NS

CHUNK = 128
NCHUNKS = 80
EPT = NCHUNKS * CHUNK
E_PAD = NW * EPT
N_PAD = 10240
ROWS_PER_TILE = N_PAD // NS

_mesh = plsc.VectorSubcoreMesh(core_axis_name="c", subcore_axis_name="s")


@functools.partial(
    pl.kernel,
    mesh=_mesh,
    out_type=jax.ShapeDtypeStruct((NC, N_PAD), jnp.float32),
    scratch_types=[
        pltpu.VMEM((2, NCHUNKS // 2, CHUNK), jnp.int32),
        pltpu.VMEM((CHUNK,), jnp.float32),
        pltpu.VMEM_SHARED((N_PAD,), jnp.float32),
    ],
)
def _sc_deg_out(srcw_hbm, ones_hbm, zeros_hbm, out_hbm, idx_s, ones_v, dego_sp):
    c = lax.axis_index("c")
    s = lax.axis_index("s")
    wid = s * NC + c
    sl = pl.ds(s * ROWS_PER_TILE, ROWS_PER_TILE)
    pltpu.sync_copy(zeros_hbm, dego_sp.at[sl])
    pltpu.sync_copy(srcw_hbm.at[wid], idx_s)
    pltpu.sync_copy(ones_hbm, ones_v)
    plsc.subcore_barrier()

    def body(i, carry):
        pltpu.sync_copy(ones_v, dego_sp.at[idx_s.at[i // (NCHUNKS // 2), i % (NCHUNKS // 2)]], add=True)
        return carry

    lax.fori_loop(0, NCHUNKS, body, 0)
    plsc.subcore_barrier()
    pltpu.sync_copy(dego_sp.at[sl], out_hbm.at[c, sl])


@functools.partial(
    pl.kernel,
    mesh=_mesh,
    out_type=(
        jax.ShapeDtypeStruct((NC, N_PAD, D), jnp.float32),
        jax.ShapeDtypeStruct((NC, N_PAD), jnp.float32),
    ),
    scratch_types=[
        pltpu.VMEM((NCHUNKS // 2, CHUNK), jnp.int32),
        pltpu.VMEM((NCHUNKS // 2, CHUNK), jnp.int32),
        pltpu.VMEM((CHUNK,), jnp.float32),
        pltpu.VMEM((CHUNK, D), jnp.float32),
        pltpu.VMEM((CHUNK, D), jnp.float32),
        pltpu.VMEM_SHARED((N_PAD, D), jnp.float32),
        pltpu.VMEM_SHARED((N_PAD,), jnp.float32),
        pltpu.SemaphoreType.DMA,
        pltpu.SemaphoreType.DMA,
    ],
)
def _sc_gather_scatter(h_hbm, srcw_hbm, dstw_hbm, ones_hbm, zeros_hbm,
                       zrows_hbm, agg_out, degi_out,
                       idx_s, idx_d, ones_v, rows0, rows1,
                       agg_sp, degi_sp, sem0, sem1):
    c = lax.axis_index("c")
    s = lax.axis_index("s")
    wid = s * NC + c
    sl = pl.ds(s * ROWS_PER_TILE, ROWS_PER_TILE)
    HALF = NCHUNKS // 2
    pltpu.sync_copy(zrows_hbm, agg_sp.at[sl])
    pltpu.sync_copy(zeros_hbm, degi_sp.at[sl])
    pltpu.sync_copy(ones_hbm, ones_v)
    plsc.subcore_barrier()

    for half in range(2):
        pltpu.sync_copy(srcw_hbm.at[wid, half], idx_s)
        pltpu.sync_copy(dstw_hbm.at[wid, half], idx_d)

        pltpu.async_copy(h_hbm.at[idx_s.at[0]], rows0, sem0)
        pltpu.async_copy(h_hbm.at[idx_s.at[1]], rows1, sem1)

        def body(j, carry):
            i0 = 2 * j
            pltpu.make_async_copy(h_hbm.at[idx_s.at[i0]], rows0, sem0).wait()
            pltpu.sync_copy(rows0, agg_sp.at[idx_d.at[i0]], add=True)
            pltpu.async_copy(h_hbm.at[idx_s.at[i0 + 2]], rows0, sem0)
            pltpu.sync_copy(ones_v, degi_sp.at[idx_d.at[i0]], add=True)
            pltpu.make_async_copy(h_hbm.at[idx_s.at[i0 + 1]], rows1, sem1).wait()
            pltpu.sync_copy(rows1, agg_sp.at[idx_d.at[i0 + 1]], add=True)
            pltpu.async_copy(h_hbm.at[idx_s.at[i0 + 3]], rows1, sem1)
            pltpu.sync_copy(ones_v, degi_sp.at[idx_d.at[i0 + 1]], add=True)
            return carry

        lax.fori_loop(0, HALF // 2 - 1, body, 0)

        pltpu.make_async_copy(h_hbm.at[idx_s.at[HALF - 2]], rows0, sem0).wait()
        pltpu.sync_copy(rows0, agg_sp.at[idx_d.at[HALF - 2]], add=True)
        pltpu.sync_copy(ones_v, degi_sp.at[idx_d.at[HALF - 2]], add=True)
        pltpu.make_async_copy(h_hbm.at[idx_s.at[HALF - 1]], rows1, sem1).wait()
        pltpu.sync_copy(rows1, agg_sp.at[idx_d.at[HALF - 1]], add=True)
        pltpu.sync_copy(ones_v, degi_sp.at[idx_d.at[HALF - 1]], add=True)

    plsc.subcore_barrier()
    pltpu.sync_copy(agg_sp.at[sl], agg_out.at[c, sl])
    pltpu.sync_copy(degi_sp.at[sl], degi_out.at[c, sl])


def _h_body(x_ref, d0_ref, d1_ref, o_ref):
    d = d0_ref[0, 0, :] + d1_ref[0, 0, :]
    norm = jnp.where(d > 0.0, lax.rsqrt(d), 0.0)
    o_ref[...] = x_ref[...] * norm[:, None]


def _tc_scale(x_pad, dego3d):
    R = 1024
    return pl.pallas_call(
        _h_body,
        grid=(N_PAD // R,),
        in_specs=[
            pl.BlockSpec((R, D), lambda i: (i, 0)),
            pl.BlockSpec((1, 1, R), lambda i: (0, 0, i)),
            pl.BlockSpec((1, 1, R), lambda i: (1, 0, i)),
        ],
        out_specs=pl.BlockSpec((R, D), lambda i: (i, 0)),
        out_shape=jax.ShapeDtypeStruct((N_PAD, D), jnp.float32),
    )(x_pad, dego3d, dego3d)


def _dense_body(a0_ref, a1_ref, d0_ref, d1_ref, x_ref, w_ref, b_ref,
                g_ref, bt_ref, o_ref):
    agg = a0_ref[0] + a1_ref[0]
    din = d0_ref[0, 0, :] + d1_ref[0, 0, :]
    norm = jnp.where(din > 0.0, lax.rsqrt(din), 0.0)
    a = agg * norm[:, None]
    o = jnp.dot(a, w_ref[...], preferred_element_type=jnp.float32) + b_ref[0]
    mu = jnp.mean(o, axis=1, keepdims=True)
    var = jnp.mean((o - mu) ** 2, axis=1, keepdims=True)
    o = (o - mu) * lax.rsqrt(var + 1e-5) * g_ref[0] + bt_ref[0]
    o_ref[...] = jnp.maximum(o, 0.0) + x_ref[...]


def _tc_dense(agg_p, degi3d, x_pad, W, b, gamma, beta):
    R = 1024
    return pl.pallas_call(
        _dense_body,
        grid=(N_PAD // R,),
        in_specs=[
            pl.BlockSpec((1, R, D), lambda i: (0, i, 0)),
            pl.BlockSpec((1, R, D), lambda i: (1, i, 0)),
            pl.BlockSpec((1, 1, R), lambda i: (0, 0, i)),
            pl.BlockSpec((1, 1, R), lambda i: (1, 0, i)),
            pl.BlockSpec((R, D), lambda i: (i, 0)),
            pl.BlockSpec((D, D), lambda i: (0, 0)),
            pl.BlockSpec((1, D), lambda i: (0, 0)),
            pl.BlockSpec((1, D), lambda i: (0, 0)),
            pl.BlockSpec((1, D), lambda i: (0, 0)),
        ],
        out_specs=pl.BlockSpec((R, D), lambda i: (i, 0)),
        out_shape=jax.ShapeDtypeStruct((N_PAD, D), jnp.float32),
    )(agg_p, agg_p, degi3d, degi3d, x_pad, W,
      b.reshape(1, D), gamma.reshape(1, D), beta.reshape(1, D))


def kernel(x, edge_index, W, b, gamma, beta):
    src = edge_index[0].astype(jnp.int32)
    dst = edge_index[1].astype(jnp.int32)
    pad_n = E_PAD - E
    pad_idx = N + (jnp.arange(pad_n, dtype=jnp.int32) % (N_PAD - N))
    src_w = jnp.concatenate([src, pad_idx]).reshape(NW, 2, NCHUNKS // 2, CHUNK)
    dst_w = jnp.concatenate([dst, pad_idx]).reshape(NW, 2, NCHUNKS // 2, CHUNK)

    ones_c = jnp.ones((CHUNK,), jnp.float32)
    zeros_r = jnp.zeros((ROWS_PER_TILE,), jnp.float32)
    zeros_rows = jnp.zeros((ROWS_PER_TILE, D), jnp.float32)

    dego = _sc_deg_out(src_w, ones_c, zeros_r)
    dego3d = dego.reshape(NC, 1, N_PAD)

    x_pad = jnp.pad(x, ((0, N_PAD - N), (0, 0)))
    h = _tc_scale(x_pad, dego3d)
    agg_p, degi = _sc_gather_scatter(h, src_w, dst_w, ones_c, zeros_r,
                                     zeros_rows)
    degi3d = degi.reshape(NC, 1, N_PAD)
    return _tc_dense(agg_p, degi3d, x_pad, W, b, gamma, beta)[:N]
